# baseline (speedup 1.0000x reference)
import functools

import jax
import jax.numpy as jnp
from jax.experimental import pallas as pl
from jax.experimental.pallas import tpu as pltpu

IN_FEATURES = 1024
NUM_HEADS = 16
HEAD_DIM = 128
TOP_K = 2
HIDDEN = 256
GROUPS = 4
HPG = NUM_HEADS // GROUPS


def _sq_kernel(x_ref, wr_ref, c_ref, t_ref, w_hbm, b_ref, o_ref,
               wcat, sem):
    for h in range(NUM_HEADS):
        pltpu.make_async_copy(
            w_hbm.at[h], wcat.at[:, pl.ds(h * HEAD_DIM, HEAD_DIM)],
            sem.at[h]).start()

    x = x_ref[...]
    wr = wr_ref[...]
    cents = c_ref[...]
    temp = t_ref[0, 0]

    z = jax.lax.dot_general(x, wr, (((1,), (1,)), ((), ())),
                            preferred_element_type=jnp.float32)
    ones = jnp.ones((HIDDEN, 8), dtype=jnp.float32)
    n2 = jax.lax.dot_general(z * z, ones, (((1,), (0,)), ((), ())),
                             preferred_element_type=jnp.float32)
    r = 1.0 / jnp.maximum(jnp.sqrt(n2[:, :1]), 1e-12)
    c_norm = cents / jnp.maximum(
        jnp.sqrt(jnp.sum(cents * cents, axis=-1, keepdims=True)), 1e-12)
    zc = jax.lax.dot_general(z, c_norm, (((1,), (1,)), ((), ())),
                             preferred_element_type=jnp.float32)
    logits = zc * (r * jnp.exp(temp))

    l1 = jnp.max(logits, axis=-1)[:, None]
    i1 = jnp.argmax(logits, axis=-1)[:, None]
    head_iota = jax.lax.broadcasted_iota(jnp.int32, logits.shape, 1)
    masked = jnp.where(head_iota == i1, -jnp.inf, logits)
    l2 = jnp.max(masked, axis=-1)[:, None]
    i2 = jnp.argmax(masked, axis=-1)[:, None]
    e2 = jnp.exp(l2 - l1)
    d = jnp.sum(jnp.exp(logits - l1), axis=-1)[:, None]
    denom = 1.0 + e2 + 1e-6 * d
    w1 = 1.0 / denom
    w2 = e2 / denom

    acc0 = jnp.zeros((x.shape[0], HEAD_DIM), dtype=jnp.float32)
    acc1 = jnp.zeros((x.shape[0], HEAD_DIM), dtype=jnp.float32)
    for g in range(GROUPS):
        for i in range(HPG):
            h = g * HPG + i
            pltpu.make_async_copy(
                w_hbm.at[h], wcat.at[:, pl.ds(h * HEAD_DIM, HEAD_DIM)],
                sem.at[h]).wait()
        cols = HPG * HEAD_DIM
        y_g = jnp.dot(x, wcat[:, pl.ds(g * cols, cols)],
                      preferred_element_type=jnp.float32)
        y_g = y_g + b_ref[0, pl.ds(g * cols, cols)][None, :]
        for i in range(HPG):
            h = g * HPG + i
            y_h = y_g[:, i * HEAD_DIM:(i + 1) * HEAD_DIM]
            m0 = jnp.where(i1 == h, w1, 0.0)
            m1 = jnp.where(i2 == h, w2, 0.0)
            acc0 = acc0 + m0 * y_h
            acc1 = acc1 + m1 * y_h
    o_ref[:, :HEAD_DIM] = acc0
    o_ref[:, HEAD_DIM:] = acc1


@functools.partial(jax.jit, static_argnames=())
def kernel(x, Wr, centroids, temperature, weight, bias):
    batch_shape = x.shape[:-1]
    x_flat = x.reshape(-1, IN_FEATURES)
    T = x_flat.shape[0]
    out = pl.pallas_call(
        _sq_kernel,
        in_specs=[
            pl.BlockSpec((T, IN_FEATURES), lambda: (0, 0)),
            pl.BlockSpec((HIDDEN, IN_FEATURES), lambda: (0, 0)),
            pl.BlockSpec((NUM_HEADS, HIDDEN), lambda: (0, 0)),
            pl.BlockSpec((1, 1), lambda: (0, 0)),
            pl.BlockSpec(memory_space=pltpu.MemorySpace.HBM),
            pl.BlockSpec((1, NUM_HEADS * HEAD_DIM), lambda: (0, 0)),
        ],
        out_specs=pl.BlockSpec((T, TOP_K * HEAD_DIM), lambda: (0, 0)),
        scratch_shapes=[
            pltpu.VMEM((IN_FEATURES, NUM_HEADS * HEAD_DIM), jnp.float32),
            pltpu.SemaphoreType.DMA((NUM_HEADS,)),
        ],
        out_shape=jax.ShapeDtypeStruct((T, TOP_K * HEAD_DIM), jnp.float32),
    )(x_flat, Wr, centroids, temperature.reshape(1, 1), weight,
      bias.reshape(1, NUM_HEADS * HEAD_DIM))
    return out.reshape(*batch_shape, TOP_K * HEAD_DIM)


# speedup vs baseline: 1.0117x; 1.0117x over previous
import functools

import jax
import jax.numpy as jnp
from jax.experimental import pallas as pl
from jax.experimental.pallas import tpu as pltpu

IN_FEATURES = 1024
NUM_HEADS = 16
HEAD_DIM = 128
TOP_K = 2
HIDDEN = 256
GROUPS = 4
HPG = NUM_HEADS // GROUPS


def _sq_kernel(x_ref, wr_ref, c_ref, t_ref, w_hbm, b_ref, o_ref,
               wcat, sem):
    for h in range(NUM_HEADS):
        pltpu.make_async_copy(
            w_hbm.at[h], wcat.at[:, pl.ds(h * HEAD_DIM, HEAD_DIM)],
            sem.at[h]).start()

    x = x_ref[...]
    wr = wr_ref[...]
    cents = c_ref[...]
    temp = t_ref[0, 0]

    z = jax.lax.dot_general(x, wr, (((1,), (1,)), ((), ())),
                            preferred_element_type=jnp.float32)
    ones = jnp.ones((HIDDEN, 8), dtype=jnp.float32)
    n2 = jax.lax.dot_general(z * z, ones, (((1,), (0,)), ((), ())),
                             preferred_element_type=jnp.float32)
    r = 1.0 / jnp.maximum(jnp.sqrt(n2[:, :1]), 1e-12)
    c_norm = cents / jnp.maximum(
        jnp.sqrt(jnp.sum(cents * cents, axis=-1, keepdims=True)), 1e-12)
    z_norm = z * r
    zc = jax.lax.dot_general(z_norm, c_norm, (((1,), (1,)), ((), ())),
                             preferred_element_type=jnp.float32)
    logits = zc * jnp.exp(temp)

    l1 = jnp.max(logits, axis=-1)[:, None]
    i1 = jnp.argmax(logits, axis=-1)[:, None]
    head_iota = jax.lax.broadcasted_iota(jnp.int32, logits.shape, 1)
    masked = jnp.where(head_iota == i1, -jnp.inf, logits)
    l2 = jnp.max(masked, axis=-1)[:, None]
    i2 = jnp.argmax(masked, axis=-1)[:, None]
    e2 = jnp.exp(l2 - l1)
    d = jnp.sum(jnp.exp(logits - l1), axis=-1)[:, None]
    denom = 1.0 + e2 + 1e-6 * d
    w1 = 1.0 / denom
    w2 = e2 / denom

    acc0 = jnp.zeros((x.shape[0], HEAD_DIM), dtype=jnp.float32)
    acc1 = jnp.zeros((x.shape[0], HEAD_DIM), dtype=jnp.float32)
    for g in range(GROUPS):
        for i in range(HPG):
            h = g * HPG + i
            pltpu.make_async_copy(
                w_hbm.at[h], wcat.at[:, pl.ds(h * HEAD_DIM, HEAD_DIM)],
                sem.at[h]).wait()
        cols = HPG * HEAD_DIM
        y_g = jnp.dot(x, wcat[:, pl.ds(g * cols, cols)],
                      preferred_element_type=jnp.float32)
        y_g = y_g + b_ref[0, pl.ds(g * cols, cols)][None, :]
        for i in range(HPG):
            h = g * HPG + i
            y_h = y_g[:, i * HEAD_DIM:(i + 1) * HEAD_DIM]
            m0 = jnp.where(i1 == h, w1, 0.0)
            m1 = jnp.where(i2 == h, w2, 0.0)
            acc0 = acc0 + m0 * y_h
            acc1 = acc1 + m1 * y_h
    o_ref[:, :HEAD_DIM] = acc0
    o_ref[:, HEAD_DIM:] = acc1


@functools.partial(jax.jit, static_argnames=())
def kernel(x, Wr, centroids, temperature, weight, bias):
    batch_shape = x.shape[:-1]
    x_flat = x.reshape(-1, IN_FEATURES)
    T = x_flat.shape[0]
    out = pl.pallas_call(
        _sq_kernel,
        in_specs=[
            pl.BlockSpec((T, IN_FEATURES), lambda: (0, 0)),
            pl.BlockSpec((HIDDEN, IN_FEATURES), lambda: (0, 0)),
            pl.BlockSpec((NUM_HEADS, HIDDEN), lambda: (0, 0)),
            pl.BlockSpec((1, 1), lambda: (0, 0)),
            pl.BlockSpec(memory_space=pltpu.MemorySpace.HBM),
            pl.BlockSpec((1, NUM_HEADS * HEAD_DIM), lambda: (0, 0)),
        ],
        out_specs=pl.BlockSpec((T, TOP_K * HEAD_DIM), lambda: (0, 0)),
        scratch_shapes=[
            pltpu.VMEM((IN_FEATURES, NUM_HEADS * HEAD_DIM), jnp.float32),
            pltpu.SemaphoreType.DMA((NUM_HEADS,)),
        ],
        out_shape=jax.ShapeDtypeStruct((T, TOP_K * HEAD_DIM), jnp.float32),
    )(x_flat, Wr, centroids, temperature.reshape(1, 1), weight,
      bias.reshape(1, NUM_HEADS * HEAD_DIM))
    return out.reshape(*batch_shape, TOP_K * HEAD_DIM)
